# Initial kernel scaffold; baseline (speedup 1.0000x reference)
import jax
import jax.numpy as jnp
from jax.experimental import pallas as pl


def kernel(x, slot_ids, layer_idx, A, B):
    raise NotImplementedError("write your pallas kernel here")


# speedup vs baseline: 2.8039x; 2.8039x over previous
import functools

import jax
import jax.numpy as jnp
from jax import lax
from jax.experimental import pallas as pl
from jax.experimental.pallas import tpu as pltpu
from jax.experimental.pallas import tpu_sc as plsc

S = 8
R = 64
SR = S * R
DIN = 2048
DOUT = 4096
T = 2048
L = 16
NC = 2
NS = 16
NW = NC * NS
TPW = T // NW
CH = TPW // L
BLK = 128
NBLK = T // BLK
IDR = 16
IDC = T // IDR


def _tc1_body(sid_ref, x_ref, a_ref, inter_ref, dst_ref, off_ref):
    i = pl.program_id(0)

    @pl.when(i == 0)
    def _():
        sid = sid_ref[...]
        rr = lax.broadcasted_iota(jnp.int32, (IDC, IDC), 0)
        cc = lax.broadcasted_iota(jnp.int32, (IDC, IDC), 1)
        U = jnp.where(rr <= cc, 1.0, 0.0)
        pr = lax.broadcasted_iota(jnp.int32, (IDR, IDR), 0)
        pc = lax.broadcasted_iota(jnp.int32, (IDR, IDR), 1)
        P = jnp.where(pc < pr, 1.0, 0.0)
        ones_col = jnp.ones((IDC, 1), jnp.float32)
        lane = lax.broadcasted_iota(jnp.int32, (1, L), 1)

        off_s = 0.0
        dst_f = jnp.zeros((IDR, IDC), jnp.float32)
        off_f = jnp.zeros((1, L), jnp.float32)
        for s in range(S):
            oh = jnp.where(sid == s, 1.0, 0.0)
            incl = jnp.dot(oh, U, preferred_element_type=jnp.float32)
            below = jnp.dot(P, oh, preferred_element_type=jnp.float32)
            row_pref = jnp.dot(below, ones_col,
                               preferred_element_type=jnp.float32)
            rank = incl - oh + row_pref
            off_f = jnp.where(lane == s, off_s, off_f)
            dst_f = dst_f + oh * (off_s + rank)
            off_s = off_s + jnp.sum(oh)
        off_f = jnp.where(lane >= S, off_s, off_f)
        dst_ref[...] = dst_f.astype(jnp.int32)
        off_ref[...] = off_f.astype(jnp.int32)

    inter_ref[...] = lax.dot_general(
        x_ref[...], a_ref[...], (((1,), (1,)), ((), ())),
        preferred_element_type=jnp.float32)


def _tc1(sid2, x, a_all):
    return pl.pallas_call(
        _tc1_body,
        grid=(NBLK,),
        in_specs=[
            pl.BlockSpec((IDR, IDC), lambda i: (0, 0)),
            pl.BlockSpec((BLK, DIN), lambda i: (i, 0)),
            pl.BlockSpec((SR, DIN), lambda i: (0, 0)),
        ],
        out_specs=[
            pl.BlockSpec((BLK, SR), lambda i: (i, 0)),
            pl.BlockSpec((IDR, IDC), lambda i: (0, 0)),
            pl.BlockSpec((1, L), lambda i: (0, 0)),
        ],
        out_shape=[
            jax.ShapeDtypeStruct((T, SR), jnp.float32),
            jax.ShapeDtypeStruct((IDR, IDC), jnp.int32),
            jax.ShapeDtypeStruct((1, L), jnp.int32),
        ],
    )(sid2, x, a_all)


@functools.partial(
    pl.kernel,
    mesh=plsc.VectorSubcoreMesh(core_axis_name="c", subcore_axis_name="s"),
    out_type=jax.ShapeDtypeStruct((T, SR), jnp.float32),
    scratch_types=[
        pltpu.VMEM((TPW,), jnp.int32),
        pltpu.VMEM((TPW, SR), jnp.float32),
        pltpu.SemaphoreType.DMA,
    ],
)
def _sc_scatter_inter(dst_hbm, inter_hbm, is_hbm, dst_v, buf, sem):
    wid = lax.axis_index("s") * NC + lax.axis_index("c")
    base = wid * TPW
    pltpu.sync_copy(dst_hbm.at[pl.ds(base, TPW)], dst_v)
    pltpu.sync_copy(inter_hbm.at[pl.ds(base, TPW)], buf)
    pltpu.async_copy(buf, is_hbm.at[dst_v], sem).wait()


def _tc2_body(off_ref, is_ref, b_ref, o_ref):
    blk = pl.program_id(0) * BLK
    rows = blk + lax.broadcasted_iota(jnp.int32, (BLK, 1), 0)
    dn = (((1,), (1,)), ((), ()))

    o_ref[...] = jnp.zeros((BLK, DOUT), jnp.float32)
    for s in range(S):
        lo = off_ref[0, s]
        hi = off_ref[0, s + 1] if s + 1 < S else T

        @pl.when(jnp.logical_and(hi > blk, lo < blk + BLK))
        def _(s=s, lo=lo, hi=hi):
            mask = jnp.logical_and(rows >= lo, rows < hi)
            ob = lax.dot_general(is_ref[:, s * R:(s + 1) * R], b_ref[s], dn,
                                 preferred_element_type=jnp.float32)
            o_ref[...] += jnp.where(mask, ob, 0.0)


def _tc2(off2, is_, B):
    return pl.pallas_call(
        _tc2_body,
        grid=(NBLK,),
        in_specs=[
            pl.BlockSpec(memory_space=pltpu.SMEM),
            pl.BlockSpec((BLK, SR), lambda i: (i, 0)),
            pl.BlockSpec((S, DOUT, R), lambda i: (0, 0, 0)),
        ],
        out_specs=pl.BlockSpec((BLK, DOUT), lambda i: (i, 0)),
        out_shape=jax.ShapeDtypeStruct((T, DOUT), jnp.float32),
    )(off2, is_, B)


@functools.partial(
    pl.kernel,
    mesh=plsc.VectorSubcoreMesh(core_axis_name="c", subcore_axis_name="s"),
    out_type=jax.ShapeDtypeStruct((T, DOUT), jnp.float32),
    scratch_types=[
        pltpu.VMEM((TPW,), jnp.int32),
        pltpu.VMEM((L, DOUT), jnp.float32),
        pltpu.SemaphoreType.DMA,
    ],
)
def _sc_unpermute(dst_hbm, os_hbm, out_hbm, dst_v, buf, sem):
    wid = lax.axis_index("s") * NC + lax.axis_index("c")
    base = wid * TPW
    pltpu.sync_copy(dst_hbm.at[pl.ds(base, TPW)], dst_v)
    for c in range(CH):
        idxr = dst_v[pl.ds(c * L, L)]
        pltpu.async_copy(os_hbm.at[idxr], buf, sem).wait()
        pltpu.sync_copy(buf, out_hbm.at[pl.ds(base + c * L, L)])


def kernel(x, slot_ids, layer_idx, A, B):
    del layer_idx
    sid2 = slot_ids.astype(jnp.int32).reshape(IDR, IDC)
    a_all = A.reshape(SR, DIN)
    inter, dst2, off2 = _tc1(sid2, x, a_all)
    dst = dst2.reshape(T)
    is_ = _sc_scatter_inter(dst, inter)
    out_sorted = _tc2(off2, is_, B)
    return _sc_unpermute(dst, out_sorted)


# speedup vs baseline: 2.8442x; 1.0144x over previous
import functools

import jax
import jax.numpy as jnp
from jax import lax
from jax.experimental import pallas as pl
from jax.experimental.pallas import tpu as pltpu
from jax.experimental.pallas import tpu_sc as plsc

S = 8
R = 64
DIN = 2048
DOUT = 4096
T = 2048
L = 16
NC = 2
NS = 16
NW = NC * NS
TPW = T // NW
CH = TPW // L
GC = 8
NG = TPW // GC
BLK = 128
NBLK = T // BLK
IDR = 16
IDC = T // IDR


def _meta_body(sid_ref, dst_ref, off_ref):
    sid = sid_ref[...]
    rr = lax.broadcasted_iota(jnp.int32, (IDC, IDC), 0)
    cc = lax.broadcasted_iota(jnp.int32, (IDC, IDC), 1)
    U = jnp.where(rr <= cc, 1.0, 0.0)
    pr = lax.broadcasted_iota(jnp.int32, (IDR, IDR), 0)
    pc = lax.broadcasted_iota(jnp.int32, (IDR, IDR), 1)
    P = jnp.where(pc < pr, 1.0, 0.0)
    ones_col = jnp.ones((IDC, 1), jnp.float32)
    lane = lax.broadcasted_iota(jnp.int32, (1, L), 1)

    off_s = 0.0
    dst_f = jnp.zeros((IDR, IDC), jnp.float32)
    off_f = jnp.zeros((1, L), jnp.float32)
    for s in range(S):
        oh = jnp.where(sid == s, 1.0, 0.0)
        incl = jnp.dot(oh, U, preferred_element_type=jnp.float32)
        below = jnp.dot(P, oh, preferred_element_type=jnp.float32)
        row_pref = jnp.dot(below, ones_col, preferred_element_type=jnp.float32)
        rank = incl - oh + row_pref
        off_f = jnp.where(lane == s, off_s, off_f)
        dst_f = dst_f + oh * (off_s + rank)
        off_s = off_s + jnp.sum(oh)
    off_f = jnp.where(lane >= S, off_s, off_f)
    dst_ref[...] = dst_f.astype(jnp.int32)
    off_ref[...] = off_f.astype(jnp.int32)


def _tc_meta(sid2):
    return pl.pallas_call(
        _meta_body,
        out_shape=[
            jax.ShapeDtypeStruct((IDR, IDC), jnp.int32),
            jax.ShapeDtypeStruct((1, L), jnp.int32),
        ],
    )(sid2)


@functools.partial(
    pl.kernel,
    mesh=plsc.VectorSubcoreMesh(core_axis_name="c", subcore_axis_name="s"),
    out_type=jax.ShapeDtypeStruct((T, DIN), jnp.float32),
    scratch_types=[
        pltpu.VMEM((TPW,), jnp.int32),
        pltpu.VMEM((L, DIN), jnp.float32),
        pltpu.VMEM((L, DIN), jnp.float32),
        pltpu.SemaphoreType.DMA,
        pltpu.SemaphoreType.DMA,
    ],
)
def _sc_scatter_x(dst_hbm, x_hbm, xs_hbm, dst_v, buf0, buf1, sem0, sem1):
    wid = lax.axis_index("s") * NC + lax.axis_index("c")
    base = wid * TPW
    pltpu.sync_copy(dst_hbm.at[pl.ds(base, TPW)], dst_v)
    bufs = (buf0, buf1)
    sems = (sem0, sem1)
    hs = [None, None]
    for c in range(CH):
        b = c & 1
        if hs[b] is not None:
            hs[b].wait()
        pltpu.sync_copy(x_hbm.at[pl.ds(base + c * L, L)], bufs[b])
        idx = dst_v[pl.ds(c * L, L)]
        hs[b] = pltpu.async_copy(bufs[b], xs_hbm.at[idx], sems[b])
    hs[0].wait()
    hs[1].wait()


def _tc_fused_body(off_ref, x_ref, a_ref, b_ref, o_ref):
    blk = pl.program_id(0) * BLK
    rows = blk + lax.broadcasted_iota(jnp.int32, (BLK, 1), 0)
    dn = (((1,), (1,)), ((), ()))

    o_ref[...] = jnp.zeros((BLK, DOUT), jnp.float32)
    for s in range(S):
        lo = off_ref[0, s]
        hi = off_ref[0, s + 1] if s + 1 < S else T

        @pl.when(jnp.logical_and(hi > blk, lo < blk + BLK))
        def _(s=s, lo=lo, hi=hi):
            mask = jnp.logical_and(rows >= lo, rows < hi)
            p = lax.dot_general(x_ref[...], a_ref[s], dn,
                                preferred_element_type=jnp.float32)
            ob = lax.dot_general(p, b_ref[s], dn,
                                 preferred_element_type=jnp.float32)
            o_ref[...] += jnp.where(mask, ob, 0.0)


def _tc_fused(off2, xs, A, B):
    return pl.pallas_call(
        _tc_fused_body,
        grid=(NBLK,),
        in_specs=[
            pl.BlockSpec(memory_space=pltpu.SMEM),
            pl.BlockSpec((BLK, DIN), lambda i: (i, 0)),
            pl.BlockSpec((S, R, DIN), lambda i: (0, 0, 0)),
            pl.BlockSpec((S, DOUT, R), lambda i: (0, 0, 0)),
        ],
        out_specs=pl.BlockSpec((BLK, DOUT), lambda i: (i, 0)),
        out_shape=jax.ShapeDtypeStruct((T, DOUT), jnp.float32),
    )(off2, xs, A, B)


@functools.partial(
    pl.kernel,
    mesh=plsc.VectorSubcoreMesh(core_axis_name="c", subcore_axis_name="s"),
    out_type=jax.ShapeDtypeStruct((T, DOUT), jnp.float32),
    scratch_types=[
        pltpu.VMEM((TPW,), jnp.int32),
        pltpu.VMEM((GC, DOUT), jnp.float32),
        pltpu.VMEM((GC, DOUT), jnp.float32),
        pltpu.SemaphoreType.DMA,
        pltpu.SemaphoreType.DMA,
    ],
)
def _sc_unpermute(dst_hbm, os_hbm, out_hbm, dst_v, buf0, buf1, sem0, sem1):
    wid = lax.axis_index("s") * NC + lax.axis_index("c")
    base = wid * TPW
    pltpu.sync_copy(dst_hbm.at[pl.ds(base, TPW)], dst_v)
    bufs = (buf0, buf1)
    sems = (sem0, sem1)
    hs = [None, None]
    for c in range(NG):
        b = c & 1
        if hs[b] is not None:
            hs[b].wait()
            pltpu.sync_copy(bufs[b], out_hbm.at[pl.ds(base + (c - 2) * GC, GC)])
        hs[b] = pltpu.async_copy(
            os_hbm.at[dst_v.at[pl.ds(c * GC, GC)]], bufs[b], sems[b])
    for c in (NG - 2, NG - 1):
        b = c & 1
        hs[b].wait()
        pltpu.sync_copy(bufs[b], out_hbm.at[pl.ds(base + c * GC, GC)])


def kernel(x, slot_ids, layer_idx, A, B):
    del layer_idx
    sid2 = slot_ids.astype(jnp.int32).reshape(IDR, IDC)
    dst2, off2 = _tc_meta(sid2)
    dst = dst2.reshape(T)
    xs = _sc_scatter_x(dst, x)
    out_sorted = _tc_fused(off2, xs, A, B)
    return _sc_unpermute(dst, out_sorted)


# speedup vs baseline: 5.4085x; 1.9016x over previous
import jax
import jax.numpy as jnp
from jax import lax
from jax.experimental import pallas as pl
from jax.experimental.pallas import tpu as pltpu

S = 8
R = 64
SR = S * R
DIN = 2048
DOUT = 4096
T = 2048
BLK = 128
NBLK = T // BLK


def _body(sid_ref, x_ref, a_ref, b_ref, o_ref, a16_ref, bt_ref):
    i = pl.program_id(0)

    @pl.when(i == 0)
    def _():
        a16_ref[...] = a_ref[...].astype(jnp.bfloat16)
        for s in range(S):
            bt_ref[pl.ds(s * R, R), :] = (
                jnp.transpose(b_ref[s]).astype(jnp.bfloat16))

    xb = x_ref[...].astype(jnp.bfloat16)
    inter = lax.dot_general(xb, a16_ref[...], (((1,), (1,)), ((), ())),
                            preferred_element_type=jnp.float32)
    band = lax.broadcasted_iota(jnp.int32, (BLK, SR), 1) // R
    mask = band == sid_ref[...]
    minter = jnp.where(mask, inter, 0.0).astype(jnp.bfloat16)
    o_ref[...] = lax.dot_general(minter, bt_ref[...], (((1,), (0,)), ((), ())),
                                 preferred_element_type=jnp.float32)


def kernel(x, slot_ids, layer_idx, A, B):
    del layer_idx
    sid_col = slot_ids.astype(jnp.int32).reshape(T, 1)
    a_all = A.reshape(SR, DIN)
    return pl.pallas_call(
        _body,
        grid=(NBLK,),
        in_specs=[
            pl.BlockSpec((BLK, 1), lambda i: (i, 0)),
            pl.BlockSpec((BLK, DIN), lambda i: (i, 0)),
            pl.BlockSpec((SR, DIN), lambda i: (0, 0)),
            pl.BlockSpec((S, DOUT, R), lambda i: (0, 0, 0)),
        ],
        out_specs=pl.BlockSpec((BLK, DOUT), lambda i: (i, 0)),
        out_shape=jax.ShapeDtypeStruct((T, DOUT), jnp.float32),
        scratch_shapes=[
            pltpu.VMEM((SR, DIN), jnp.bfloat16),
            pltpu.VMEM((SR, DOUT), jnp.bfloat16),
        ],
    )(sid_col, x, a_all, B)
